# Initial kernel scaffold; baseline (speedup 1.0000x reference)
import jax
import jax.numpy as jnp
from jax.experimental import pallas as pl


def kernel(sparse_repr):
    raise NotImplementedError("write your pallas kernel here")


# speedup vs baseline: 1.4732x; 1.4732x over previous
import functools

import jax
import jax.numpy as jnp
from jax import lax
from jax.experimental import pallas as pl
from jax.experimental.pallas import tpu as pltpu
from jax.experimental.pallas import tpu_sc as plsc

ROWS = 1024
COLS = 100000
TOPK = 5
MINACT = 0.5
LANES = 16
NCHAIN = 5
VPR = COLS // LANES
ITERS = VPR // NCHAIN
NEG = -3.0e38


def _insert(ts, x):
    t0, t1, t2, t3, t4 = ts
    y = jnp.minimum(t0, x)
    t0 = jnp.maximum(t0, x)
    y2 = jnp.minimum(t1, y)
    t1 = jnp.maximum(t1, y)
    y3 = jnp.minimum(t2, y2)
    t2 = jnp.maximum(t2, y2)
    y4 = jnp.minimum(t3, y3)
    t3 = jnp.maximum(t3, y3)
    t4 = jnp.maximum(t4, y4)
    return (t0, t1, t2, t3, t4)


def _permute(x, idx):
    dnums = lax.GatherDimensionNumbers(
        offset_dims=(), collapsed_slice_dims=(0,), start_index_map=(0,))
    return lax.gather(x, idx[:, None], dnums, slice_sizes=(1,),
                      mode=lax.GatherScatterMode.PROMISE_IN_BOUNDS)


def _lane_reduce_splat(x, op):
    iot = lax.iota(jnp.int32, LANES)
    for sh in (8, 4, 2, 1):
        x = op(x, _permute(x, iot ^ sh))
    return x


def _row_loss(ts):
    t0, t1, t2, t3, t4 = ts
    iot = lax.iota(jnp.int32, LANES)
    acc = jnp.zeros((LANES,), jnp.float32)
    for _ in range(TOPK):
        gs = _lane_reduce_splat(t0, jnp.maximum)
        acc = acc + gs
        cand = jnp.where(t0 == gs, iot, LANES)
        fs = _lane_reduce_splat(cand, jnp.minimum)
        pm = iot == fs
        t0 = jnp.where(pm, t1, t0)
        t1 = jnp.where(pm, t2, t1)
        t2 = jnp.where(pm, t3, t2)
        t3 = jnp.where(pm, t4, t3)
        t4 = jnp.where(pm, jnp.float32(NEG), t4)
    mean5 = acc * jnp.float32(1.0 / TOPK)
    return jnp.maximum(jnp.float32(MINACT) - mean5, 0.0)


def _sc_body(x_hbm, out_hbm, buf, part_v):
    c = lax.axis_index("c")
    s = lax.axis_index("s")
    wid = s * 2 + c
    rows_per_w = ROWS // 32
    neg = jnp.full((LANES,), NEG, jnp.float32)

    def row_body(r_local, part):
        r = wid * rows_per_w + r_local
        pltpu.sync_copy(x_hbm.at[r], buf)

        init = tuple(tuple(neg for _ in range(TOPK)) for _ in range(NCHAIN))

        def scan_body(i, chains):
            base = i * (NCHAIN * LANES)
            out = []
            for j in range(NCHAIN):
                x = buf[pl.ds(base + j * LANES, LANES)]
                out.append(_insert(chains[j], x))
            return tuple(out)

        chains = lax.fori_loop(0, ITERS, scan_body, init)

        merged = chains[0]
        for j in range(1, NCHAIN):
            for v in chains[j]:
                merged = _insert(merged, v)

        return part + _row_loss(merged)

    part = lax.fori_loop(0, rows_per_w, row_body,
                         jnp.zeros((LANES,), jnp.float32))
    part_v[...] = part
    pltpu.sync_copy(part_v, out_hbm.at[pl.ds(wid * LANES, LANES)])


def _final_reduce_body(x_ref, o_ref):
    s = jnp.sum(x_ref[...]) * (1.0 / (LANES * ROWS))
    o_ref[...] = jnp.reshape(s, (1, 1))


def kernel(sparse_repr):
    mesh = plsc.VectorSubcoreMesh(core_axis_name="c", subcore_axis_name="s")
    sc_call = functools.partial(
        pl.kernel,
        mesh=mesh,
        out_type=jax.ShapeDtypeStruct((32 * LANES,), jnp.float32),
        scratch_types=[
            pltpu.VMEM((COLS,), jnp.float32),
            pltpu.VMEM((LANES,), jnp.float32),
        ],
    )(_sc_body)
    partials = sc_call(sparse_repr)

    res = pl.pallas_call(
        _final_reduce_body,
        out_shape=jax.ShapeDtypeStruct((1, 1), jnp.float32),
    )(partials.reshape(1, 32 * LANES))
    return res[0, 0]
